# baseline (speedup 1.0000x reference)
import functools

import jax
import jax.numpy as jnp
from jax import lax
from jax.experimental import pallas as pl
from jax.experimental.pallas import tpu as pltpu
from jax.experimental.pallas import tpu_sc as plsc

MASKID = 0
CHUNK = 256
NBUF = 3


@functools.lru_cache(maxsize=None)
def _build(total, vocab, dim):
    info = plsc.get_sparse_core_info()
    nw = info.num_cores * info.num_subcores
    per_w = total // nw
    n_chunks = per_w // CHUNK
    assert per_w * nw == total and n_chunks * CHUNK == per_w
    assert n_chunks >= NBUF + 2

    mesh = plsc.VectorSubcoreMesh(core_axis_name="c", subcore_axis_name="s")

    @functools.partial(
        pl.kernel,
        mesh=mesh,
        out_type=(
            jax.ShapeDtypeStruct((total, dim), jnp.float32),
            jax.ShapeDtypeStruct((total,), jnp.int32),
        ),
        scratch_types=[
            pltpu.VMEM((per_w,), jnp.int32),
            pltpu.VMEM((per_w,), jnp.int32),
            [pltpu.VMEM((CHUNK, dim), jnp.float32) for _ in range(NBUF)],
            [pltpu.SemaphoreType.DMA for _ in range(NBUF)],
            [pltpu.SemaphoreType.DMA for _ in range(NBUF)],
        ],
    )
    def emb(x_hbm, table_hbm, out_hbm, mask_hbm,
            idx_v, mask_v, bufs, isems, osems):
        wid = lax.axis_index("s") * info.num_cores + lax.axis_index("c")
        base = wid * per_w

        pltpu.sync_copy(x_hbm.at[pl.ds(base, per_w)], idx_v)

        def gather(c, b):
            pltpu.async_copy(
                table_hbm.at[idx_v.at[pl.ds(c * CHUNK, CHUNK)]],
                bufs[b], isems[b],
            )

        def drain_in(c, b):
            pltpu.make_async_copy(
                table_hbm.at[idx_v.at[pl.ds(c * CHUNK, CHUNK)]],
                bufs[b], isems[b],
            ).wait()

        def copyout(c, b):
            pltpu.async_copy(
                bufs[b], out_hbm.at[pl.ds(base + c * CHUNK, CHUNK)], osems[b]
            )

        def drain_out(c, b):
            pltpu.make_async_copy(
                bufs[b], out_hbm.at[pl.ds(base + c * CHUNK, CHUNK)], osems[b]
            ).wait()

        gather(0, 0)
        gather(1, 1)

        def mask_body(i, _):
            for j in range(CHUNK // 16):
                o = i * CHUNK + j * 16
                v = idx_v[pl.ds(o, 16)]
                mask_v[pl.ds(o, 16)] = jnp.where(v != MASKID, 1, 0).astype(
                    jnp.int32
                )
            return 0

        lax.fori_loop(0, n_chunks, mask_body, 0)
        pltpu.sync_copy(mask_v, mask_hbm.at[pl.ds(base, per_w)])

        drain_in(0, 0)
        copyout(0, 0)
        gather(2, 2)
        drain_in(1, 1)
        copyout(1, 1)
        drain_out(0, 0)
        gather(3, 0)

        def body(g, _):
            for b in range(NBUF):
                c = g * NBUF + b + 2
                bc = (b + 2) % NBUF
                drain_in(c, bc)
                copyout(c, bc)
                cg = c + 2
                bg = (b + 1) % NBUF
                drain_out(cg - NBUF, bg)
                gather(cg, bg)
            return 0

        lax.fori_loop(0, (n_chunks - 4) // NBUF, body, 0)

        drain_in(n_chunks - 2, (n_chunks - 2) % NBUF)
        copyout(n_chunks - 2, (n_chunks - 2) % NBUF)
        drain_in(n_chunks - 1, (n_chunks - 1) % NBUF)
        copyout(n_chunks - 1, (n_chunks - 1) % NBUF)

        for c in range(n_chunks - NBUF, n_chunks):
            drain_out(c, c % NBUF)

    return emb


def kernel(x, table):
    bsz, seq = x.shape
    vocab, dim = table.shape
    total = bsz * seq
    xf = x.reshape(total).astype(jnp.int32)
    emb = _build(total, vocab, dim)
    out, mask = emb(xf, table)
    return out.reshape(bsz, seq, dim), mask.reshape(bsz, seq)


# speedup vs baseline: 1.6399x; 1.6399x over previous
import functools

import jax
import jax.numpy as jnp
from jax import lax
from jax.experimental import pallas as pl
from jax.experimental.pallas import tpu as pltpu
from jax.experimental.pallas import tpu_sc as plsc

MASKID = 0
NBUF = 4
LOOKAHEAD = 2


@functools.lru_cache(maxsize=None)
def _build(bsz, seq, vocab, dim):
    info = plsc.get_sparse_core_info()
    nw = info.num_cores * info.num_subcores
    rows_w = bsz // nw
    assert rows_w * nw == bsz
    assert rows_w % NBUF == 0 and LOOKAHEAD < NBUF

    mesh = plsc.VectorSubcoreMesh(core_axis_name="c", subcore_axis_name="s")

    @functools.partial(
        pl.kernel,
        mesh=mesh,
        out_type=jax.ShapeDtypeStruct((bsz, seq, dim), jnp.float32),
        scratch_types=[
            pltpu.VMEM((rows_w, seq), jnp.int32),
            [pltpu.VMEM((seq, dim), jnp.float32) for _ in range(NBUF)],
            [pltpu.SemaphoreType.DMA for _ in range(NBUF)],
            [pltpu.SemaphoreType.DMA for _ in range(NBUF)],
        ],
    )
    def emb(x_hbm, table_hbm, out_hbm, idx_v, bufs, isems, osems):
        wid = lax.axis_index("s") * info.num_cores + lax.axis_index("c")
        row0 = wid * rows_w

        pltpu.sync_copy(x_hbm.at[pl.ds(row0, rows_w)], idx_v)

        def gather(r, b):
            pltpu.async_copy(
                table_hbm.at[idx_v.at[r]], bufs[b], isems[b]
            )

        def drain_in(r, b):
            pltpu.make_async_copy(
                table_hbm.at[idx_v.at[r]], bufs[b], isems[b]
            ).wait()

        def copyout(r, b):
            pltpu.async_copy(bufs[b], out_hbm.at[row0 + r], osems[b])

        def drain_out(r, b):
            pltpu.make_async_copy(
                bufs[b], out_hbm.at[row0 + r], osems[b]
            ).wait()

        for r in range(LOOKAHEAD):
            gather(r, r % NBUF)

        for b in range(NBUF):
            r = b
            drain_in(r, b)
            copyout(r, b)
            rg = r + LOOKAHEAD
            bg = rg % NBUF
            if rg >= NBUF:
                drain_out(rg - NBUF, bg)
            gather(rg, bg)

        def body(g, _):
            for b in range(NBUF):
                r = g * NBUF + b
                drain_in(r, b)
                copyout(r, b)
                rg = r + LOOKAHEAD
                bg = (b + LOOKAHEAD) % NBUF

                @pl.when(rg < rows_w)
                def _():
                    drain_out(rg - NBUF, bg)
                    gather(rg, bg)

            return 0

        lax.fori_loop(1, rows_w // NBUF, body, 0)

        for b in range(NBUF):
            drain_out(rows_w - NBUF + b, b)

    return emb


def _mask_body(x_ref, mask_ref):
    mask_ref[...] = jnp.where(x_ref[...] != MASKID, 1, 0).astype(jnp.int32)


@functools.lru_cache(maxsize=None)
def _build_mask(bsz, seq):
    return pl.pallas_call(
        _mask_body,
        out_shape=jax.ShapeDtypeStruct((bsz, seq), jnp.int32),
    )


def kernel(x, table):
    bsz, seq = x.shape
    vocab, dim = table.shape
    xi = x.astype(jnp.int32)
    out = _build(bsz, seq, vocab, dim)(xi, table)
    mask = _build_mask(bsz, seq)(xi)
    return out, mask
